# Initial kernel scaffold; baseline (speedup 1.0000x reference)
import jax
import jax.numpy as jnp
from jax.experimental import pallas as pl


def kernel(x, e, hyperedge_index, node_types, hyperedge_types, ln_gamma, ln_beta, W, b):
    raise NotImplementedError("write your pallas kernel here")


# speedup vs baseline: 4.5066x; 4.5066x over previous
import dataclasses
import functools

import jax
import jax.numpy as jnp
from jax import lax
from jax.experimental import pallas as pl
from jax.experimental.pallas import tpu as pltpu
from jax.experimental.pallas import tpu_sc as plsc

_D = 4
_TW = 6
_EPS = 1e-5
_L = 16
_NC, _NS = 2, 16
_NW = _NC * _NS


def _tables_body(x_ref, e_ref, wx_ref, we_ref, tx_ref, te_ref):
    df = wx_ref.shape[0]

    def one(src, w, dst):
        y = src[...]
        m = (y[:, 0:df] + y[:, df:2 * df] + y[:, 2 * df:3 * df]
             + y[:, 3 * df:4 * df]) * 0.25
        a = jnp.dot(m, w[...], preferred_element_type=jnp.float32,
                    precision=lax.Precision.HIGHEST)
        q = jnp.sum(m * m, axis=1, keepdims=True)
        colid = lax.broadcasted_iota(jnp.int32, a.shape, 1)
        dst[...] = a + jnp.where(colid == _TW - 1, q, 0.0)

    one(x_ref, wx_ref, tx_ref)
    one(e_ref, we_ref, te_ref)


def _build_tables(x4, e4, wx, we):
    nn = x4.shape[0]
    ne = e4.shape[0]
    df = wx.shape[0]
    bn = 2000
    grid = (nn // bn,)
    return pl.pallas_call(
        _tables_body,
        grid=grid,
        in_specs=[
            pl.BlockSpec((bn, 4 * df), lambda i: (i, 0)),
            pl.BlockSpec((bn, 4 * df), lambda i: (i, 0)),
            pl.BlockSpec((df, _TW), lambda i: (0, 0)),
            pl.BlockSpec((df, _TW), lambda i: (0, 0)),
        ],
        out_specs=[
            pl.BlockSpec((bn, _TW), lambda i: (i, 0)),
            pl.BlockSpec((bn, _TW), lambda i: (i, 0)),
        ],
        out_shape=[
            jax.ShapeDtypeStruct((nn, _TW), jnp.float32),
            jax.ShapeDtypeStruct((ne, _TW), jnp.float32),
        ],
    )(x4, e4, wx, we)


def _sc_combine(tx, te, row, col, c0):
    nn = tx.shape[0]
    ne = te.shape[0]
    E = row.shape[0]
    share = E // _NW
    C = 400
    txf = jnp.concatenate(
        [tx.reshape(nn * _TW), c0, jnp.zeros((4,), jnp.float32)])
    tef = te.reshape(ne * _TW)
    mesh = plsc.VectorSubcoreMesh(core_axis_name="c", subcore_axis_name="s")
    cp = pltpu.CompilerParams()
    if "needs_layout_passes" in pltpu.CompilerParams.__dataclass_fields__:
        cp = dataclasses.replace(cp, needs_layout_passes=False)

    @functools.partial(
        pl.kernel,
        compiler_params=cp,
        out_type=jax.ShapeDtypeStruct((E * _D,), jnp.float32),
        mesh=mesh,
        scratch_types=[
            pltpu.VMEM((nn * _TW + 8,), jnp.float32),
            pltpu.VMEM((ne * _TW,), jnp.float32),
            pltpu.VMEM((C,), jnp.int32),
            pltpu.VMEM((C,), jnp.int32),
            pltpu.VMEM((C * _D,), jnp.float32),
        ],
    )
    def sc_kernel(tx_hbm, te_hbm, row_hbm, col_hbm, out_hbm,
                  txv, tev, rowv, colv, outv):
        wid = lax.axis_index("s") * _NC + lax.axis_index("c")
        base = wid * share
        pltpu.sync_copy(tx_hbm, txv)
        pltpu.sync_copy(te_hbm, tev)

        lane = lax.iota(jnp.int32, _L)

        @pl.loop(0, share, step=C)
        def _chunk(t):
            pltpu.sync_copy(row_hbm.at[pl.ds(base + t, C)], rowv)
            pltpu.sync_copy(col_hbm.at[pl.ds(base + t, C)], colv)

            @pl.loop(0, C, step=_L)
            def _vec(j):
                rb = rowv[pl.ds(j, _L)] * _TW
                cb = colv[pl.ds(j, _L)] * _TW

                def g(tab, ids, k):
                    return plsc.load_gather(tab, [ids + k])

                p = [g(txv, rb, k) + g(tev, cb, k) for k in range(_D)]
                ssum = g(txv, rb, 4) + g(tev, cb, 4)
                qsum = g(txv, rb, 5) + g(tev, cb, 5)
                mu = ssum * (1.0 / 256.0)
                v = qsum * (1.0 / 256.0) - mu * mu + _EPS
                iv = plsc.bitcast(v, jnp.int32)
                y = plsc.bitcast(jnp.int32(0x5F3759DF) - (iv >> 1), jnp.float32)
                y = y * (1.5 - 0.5 * v * y * y)
                y = y * (1.5 - 0.5 * v * y * y)
                y = y * (1.5 - 0.5 * v * y * y)
                ob = (lane + j) * _D
                for k in range(_D):
                    c0k = plsc.load_gather(
                        txv, [jnp.full((_L,), nn * _TW + k, jnp.int32)])
                    z = p[k] * y + c0k
                    o = 1.0 / (1.0 + jnp.exp(-z))
                    plsc.store_scatter(outv, [ob + k], o)

            pltpu.sync_copy(outv, out_hbm.at[pl.ds((base + t) * _D, C * _D)])

    return sc_kernel(txf, tef, row, col).reshape(E, _D)


def kernel(x, e, hyperedge_index, node_types, hyperedge_types,
           ln_gamma, ln_beta, W, b):
    df = x.shape[-1]
    nn = x.shape[0] // _D
    ne = e.shape[0] // _D
    Wg = ln_gamma[:, None] * W
    gw = ln_gamma @ W
    c0 = ln_beta @ W + b
    ones = jnp.ones((df, 1), jnp.float32)
    zeros = jnp.zeros((df, 1), jnp.float32)
    wx = jnp.concatenate([Wg[:df] - gw[None, :] * (1.0 / 256.0), ones, zeros],
                         axis=1)
    we = jnp.concatenate([Wg[df:] - gw[None, :] * (1.0 / 256.0), ones, zeros],
                         axis=1)
    x4 = x.reshape(nn, _D * df)
    e4 = e.reshape(ne, _D * df)
    tx, te = _build_tables(x4, e4, wx, we)
    hi = hyperedge_index.astype(jnp.int32)
    return _sc_combine(tx, te, hi[0], hi[1], c0)


# speedup vs baseline: 10.1069x; 2.2427x over previous
import dataclasses
import functools

import jax
import jax.numpy as jnp
from jax import lax
from jax.experimental import pallas as pl
from jax.experimental.pallas import tpu as pltpu
from jax.experimental.pallas import tpu_sc as plsc

_D = 4
_TW = 6
_EPS = 1e-5
_L = 16
_NC, _NS = 2, 16
_NW = _NC * _NS


def _tables_body(x_ref, e_ref, wx_ref, we_ref, tx_ref, te_ref):
    df = wx_ref.shape[0]
    pid = pl.program_id(0)

    def one(src, w, dst):
        y = src[...]
        bn = y.shape[0] // _D
        m = (y.reshape(bn, _D, df).sum(axis=1)) * 0.25
        a = jnp.dot(m, w[...], preferred_element_type=jnp.float32,
                    precision=lax.Precision.HIGHEST)
        q = jnp.sum(m * m, axis=1, keepdims=True)
        colid = lax.broadcasted_iota(jnp.int32, a.shape, 1)
        dst[pl.ds(pid * bn, bn), :] = a + jnp.where(colid == _TW - 1, q, 0.0)

    one(x_ref, wx_ref, tx_ref)
    one(e_ref, we_ref, te_ref)


def _build_tables(x, e, wx, we):
    nn = x.shape[0] // _D
    ne = e.shape[0] // _D
    df = wx.shape[0]
    bn = 2000
    grid = (nn // bn,)
    return pl.pallas_call(
        _tables_body,
        grid=grid,
        in_specs=[
            pl.BlockSpec((bn * _D, df), lambda i: (i, 0)),
            pl.BlockSpec((bn * _D, df), lambda i: (i, 0)),
            pl.BlockSpec((df, _TW), lambda i: (0, 0)),
            pl.BlockSpec((df, _TW), lambda i: (0, 0)),
        ],
        out_specs=[
            pl.BlockSpec((nn, _TW), lambda i: (0, 0)),
            pl.BlockSpec((ne, _TW), lambda i: (0, 0)),
        ],
        out_shape=[
            jax.ShapeDtypeStruct((nn, _TW), jnp.float32),
            jax.ShapeDtypeStruct((ne, _TW), jnp.float32),
        ],
    )(x, e, wx, we)


def _sc_combine(tx, te, row, col, c0):
    nn = tx.shape[0]
    ne = te.shape[0]
    E = row.shape[0]
    share = E // _NW
    C = 400
    txf = jnp.concatenate(
        [tx.reshape(nn * _TW), c0, jnp.zeros((4,), jnp.float32)])
    tef = te.reshape(ne * _TW)
    mesh = plsc.VectorSubcoreMesh(core_axis_name="c", subcore_axis_name="s")
    cp = pltpu.CompilerParams()
    if "needs_layout_passes" in pltpu.CompilerParams.__dataclass_fields__:
        cp = dataclasses.replace(cp, needs_layout_passes=False)

    @functools.partial(
        pl.kernel,
        compiler_params=cp,
        out_type=jax.ShapeDtypeStruct((E * _D,), jnp.float32),
        mesh=mesh,
        scratch_types=[
            pltpu.VMEM((nn * _TW + 8,), jnp.float32),
            pltpu.VMEM((ne * _TW,), jnp.float32),
            pltpu.VMEM((C,), jnp.int32),
            pltpu.VMEM((C,), jnp.int32),
            pltpu.VMEM((C * _D,), jnp.float32),
        ],
    )
    def sc_kernel(tx_hbm, te_hbm, row_hbm, col_hbm, out_hbm,
                  txv, tev, rowv, colv, outv):
        wid = lax.axis_index("s") * _NC + lax.axis_index("c")
        base = wid * share
        pltpu.sync_copy(tx_hbm, txv)
        pltpu.sync_copy(te_hbm, tev)

        lane = lax.iota(jnp.int32, _L)

        @pl.loop(0, share, step=C)
        def _chunk(t):
            pltpu.sync_copy(row_hbm.at[pl.ds(base + t, C)], rowv)
            pltpu.sync_copy(col_hbm.at[pl.ds(base + t, C)], colv)

            @pl.loop(0, C, step=_L)
            def _vec(j):
                r = rowv[pl.ds(j, _L)]
                c = colv[pl.ds(j, _L)]

                rb = r * _TW
                cb = c * _TW

                def g(tab, ids, k):
                    return plsc.load_gather(tab, [ids + k])

                p = [g(txv, rb, k) + g(tev, cb, k) for k in range(_D)]
                ssum = g(txv, rb, 4) + g(tev, cb, 4)
                qsum = g(txv, rb, 5) + g(tev, cb, 5)
                mu = ssum * (1.0 / 256.0)
                v = qsum * (1.0 / 256.0) - mu * mu + _EPS
                iv = plsc.bitcast(v, jnp.int32)
                y = plsc.bitcast(jnp.int32(0x5F3759DF) - (iv >> 1), jnp.float32)
                y = y * (1.5 - 0.5 * v * y * y)
                y = y * (1.5 - 0.5 * v * y * y)
                y = y * (1.5 - 0.5 * v * y * y)
                for k in range(_D):
                    c0k = plsc.load_gather(
                        txv, [jnp.full((_L,), nn * _TW + k, jnp.int32)])
                    z = p[k] * y + c0k
                    o = 1.0 / (1.0 + jnp.exp(-z))
                    outv[pl.ds(k * C + j, _L)] = o

            for k in range(_D):
                pltpu.sync_copy(outv.at[pl.ds(k * C, C)],
                                out_hbm.at[pl.ds(k * E + base + t, C)])

    return sc_kernel(txf, tef, row, col).reshape(_D, E).T


def kernel(x, e, hyperedge_index, node_types, hyperedge_types,
           ln_gamma, ln_beta, W, b):
    df = x.shape[-1]
    Wg = ln_gamma[:, None] * W
    gw = ln_gamma @ W
    c0 = ln_beta @ W + b
    ones = jnp.ones((df, 1), jnp.float32)
    zeros = jnp.zeros((df, 1), jnp.float32)
    wx = jnp.concatenate([Wg[:df] - gw[None, :] * (1.0 / 256.0), ones, zeros],
                         axis=1)
    we = jnp.concatenate([Wg[df:] - gw[None, :] * (1.0 / 256.0), ones, zeros],
                         axis=1)
    tx, te = _build_tables(x, e, wx, we)
    hi = hyperedge_index.astype(jnp.int32)
    return _sc_combine(tx, te, hi[0], hi[1], c0)


# speedup vs baseline: 11.8431x; 1.1718x over previous
import dataclasses
import functools

import jax
import jax.numpy as jnp
from jax import lax
from jax.experimental import pallas as pl
from jax.experimental.pallas import tpu as pltpu
from jax.experimental.pallas import tpu_sc as plsc

_D = 4
_TW = 6
_EPS = 1e-5
_L = 16
_NC, _NS = 2, 16
_NW = _NC * _NS


def _tables_body(x_ref, e_ref, wx_ref, we_ref, tx_ref, te_ref):
    df = wx_ref.shape[0]
    pid = pl.program_id(0)

    def one(src, w, dst):
        y = src[...]
        bn = y.shape[0] // _D
        m = (y.reshape(bn, _D, df).sum(axis=1)) * 0.25
        a = jnp.dot(m, w[...], preferred_element_type=jnp.float32,
                    precision=lax.Precision.HIGHEST)
        q = jnp.sum(m * m, axis=1, keepdims=True)
        colid = lax.broadcasted_iota(jnp.int32, a.shape, 1)
        dst[pl.ds(pid * bn, bn), :] = a + jnp.where(colid == _TW - 1, q, 0.0)

    one(x_ref, wx_ref, tx_ref)
    one(e_ref, we_ref, te_ref)


def _build_tables(x, e, wx, we):
    nn = x.shape[0] // _D
    ne = e.shape[0] // _D
    df = wx.shape[0]
    bn = 2000
    grid = (nn // bn,)
    return pl.pallas_call(
        _tables_body,
        grid=grid,
        in_specs=[
            pl.BlockSpec((bn * _D, df), lambda i: (i, 0)),
            pl.BlockSpec((bn * _D, df), lambda i: (i, 0)),
            pl.BlockSpec((df, _TW), lambda i: (0, 0)),
            pl.BlockSpec((df, _TW), lambda i: (0, 0)),
        ],
        out_specs=[
            pl.BlockSpec((nn, _TW), lambda i: (0, 0)),
            pl.BlockSpec((ne, _TW), lambda i: (0, 0)),
        ],
        out_shape=[
            jax.ShapeDtypeStruct((nn, _TW), jnp.float32),
            jax.ShapeDtypeStruct((ne, _TW), jnp.float32),
        ],
    )(x, e, wx, we)


def _sc_combine(tx, te, row, col, c0):
    nn = tx.shape[0]
    ne = te.shape[0]
    E = row.shape[0]
    C = 256
    txf = jnp.concatenate(
        [tx.reshape(nn * _TW), c0, jnp.zeros((4,), jnp.float32)])
    tef = te.reshape(ne * _TW)
    mesh = plsc.VectorSubcoreMesh(core_axis_name="c", subcore_axis_name="s")
    cp = pltpu.CompilerParams()
    if "needs_layout_passes" in pltpu.CompilerParams.__dataclass_fields__:
        cp = dataclasses.replace(cp, needs_layout_passes=False)

    @functools.partial(
        pl.kernel,
        compiler_params=cp,
        out_type=jax.ShapeDtypeStruct((_D, E), jnp.float32),
        mesh=mesh,
        scratch_types=[
            pltpu.VMEM((nn * _TW + 8,), jnp.float32),
            pltpu.VMEM((ne * _TW,), jnp.float32),
        ],
    )
    def sc_kernel(tx_hbm, te_hbm, row_hbm, col_hbm, out_hbm, txv, tev):
        pltpu.sync_copy(tx_hbm, txv)
        pltpu.sync_copy(te_hbm, tev)

        def body(rowv, colv, outv):
            @pl.loop(0, C, step=_L)
            def _vec(j):
                rb = rowv[pl.ds(j, _L)] * _TW
                cb = colv[pl.ds(j, _L)] * _TW

                def g(tab, ids, k):
                    return plsc.load_gather(tab, [ids + k])

                p = [g(txv, rb, k) + g(tev, cb, k) for k in range(_D)]
                ssum = g(txv, rb, 4) + g(tev, cb, 4)
                qsum = g(txv, rb, 5) + g(tev, cb, 5)
                mu = ssum * (1.0 / 256.0)
                v = qsum * (1.0 / 256.0) - mu * mu + _EPS
                iv = plsc.bitcast(v, jnp.int32)
                y = plsc.bitcast(jnp.int32(0x5F3759DF) - (iv >> 1), jnp.float32)
                y = y * (1.5 - 0.5 * v * y * y)
                y = y * (1.5 - 0.5 * v * y * y)
                y = y * (1.5 - 0.5 * v * y * y)
                for k in range(_D):
                    c0k = plsc.load_gather(
                        txv, [jnp.full((_L,), nn * _TW + k, jnp.int32)])
                    z = p[k] * y + c0k
                    o = 1.0 / (1.0 + jnp.exp(-z))
                    outv[k, pl.ds(j, _L)] = o

        pltpu.emit_pipeline(
            body,
            grid=(E // C,),
            in_specs=[pl.BlockSpec((C,), lambda i: (i,)),
                      pl.BlockSpec((C,), lambda i: (i,))],
            out_specs=[pl.BlockSpec((_D, C), lambda i: (0, i))],
            core_axis_name=("c", "s"),
            dimension_semantics=(pltpu.PARALLEL,),
        )(row_hbm, col_hbm, out_hbm)

    return sc_kernel(txf, tef, row, col).T


def kernel(x, e, hyperedge_index, node_types, hyperedge_types,
           ln_gamma, ln_beta, W, b):
    df = x.shape[-1]
    Wg = ln_gamma[:, None] * W
    gw = ln_gamma @ W
    c0 = ln_beta @ W + b
    ones = jnp.ones((df, 1), jnp.float32)
    zeros = jnp.zeros((df, 1), jnp.float32)
    wx = jnp.concatenate([Wg[:df] - gw[None, :] * (1.0 / 256.0), ones, zeros],
                         axis=1)
    we = jnp.concatenate([Wg[df:] - gw[None, :] * (1.0 / 256.0), ones, zeros],
                         axis=1)
    tx, te = _build_tables(x, e, wx, we)
    hi = hyperedge_index.astype(jnp.int32)
    return _sc_combine(tx, te, hi[0], hi[1], c0)


# speedup vs baseline: 13.8577x; 1.1701x over previous
import dataclasses
import functools

import jax
import jax.numpy as jnp
from jax import lax
from jax.experimental import pallas as pl
from jax.experimental.pallas import tpu as pltpu
from jax.experimental.pallas import tpu_sc as plsc

_D = 4
_TW = 6
_EPS = 1e-5
_L = 16
_NC, _NS = 2, 16
_NW = _NC * _NS


def _tables_body(x_ref, e_ref, wx_ref, we_ref, hi_ref, tx_ref, te_ref,
                 row_ref, col_ref):
    df = wx_ref.shape[0]
    pid = pl.program_id(0)
    hi = hi_ref[...]
    row_ref[...] = hi[0, :][None, None, :]
    col_ref[...] = hi[1, :][None, None, :]

    def one(src, w, dst):
        m = (src[0::_D, :] + src[1::_D, :] + src[2::_D, :]
             + src[3::_D, :]) * 0.25
        a = jnp.dot(m, w[...], preferred_element_type=jnp.float32,
                    precision=lax.Precision.HIGHEST)
        q = jnp.sum(m * m, axis=1, keepdims=True)
        colid = lax.broadcasted_iota(jnp.int32, a.shape, 1)
        bn = a.shape[0]
        dst[pl.ds(pid * bn, bn), :] = a + jnp.where(colid == _TW - 1, q, 0.0)

    one(x_ref, wx_ref, tx_ref)
    one(e_ref, we_ref, te_ref)


def _build_tables(x, e, wx, we, hi):
    nn = x.shape[0] // _D
    ne = e.shape[0] // _D
    E = hi.shape[1]
    df = wx.shape[0]
    bn = 2000
    grid = (nn // bn,)
    be = E // grid[0]
    return pl.pallas_call(
        _tables_body,
        grid=grid,
        in_specs=[
            pl.BlockSpec((bn * _D, df), lambda i: (i, 0)),
            pl.BlockSpec((bn * _D, df), lambda i: (i, 0)),
            pl.BlockSpec((df, _TW), lambda i: (0, 0)),
            pl.BlockSpec((df, _TW), lambda i: (0, 0)),
            pl.BlockSpec((2, be), lambda i: (0, i)),
        ],
        out_specs=[
            pl.BlockSpec((nn, _TW), lambda i: (0, 0)),
            pl.BlockSpec((ne, _TW), lambda i: (0, 0)),
            pl.BlockSpec((1, 1, be), lambda i: (i, 0, 0)),
            pl.BlockSpec((1, 1, be), lambda i: (i, 0, 0)),
        ],
        out_shape=[
            jax.ShapeDtypeStruct((nn, _TW), jnp.float32),
            jax.ShapeDtypeStruct((ne, _TW), jnp.float32),
            jax.ShapeDtypeStruct((grid[0], 1, be), jnp.int32),
            jax.ShapeDtypeStruct((grid[0], 1, be), jnp.int32),
        ],
    )(x, e, wx, we, hi)


def _sc_combine(tx, te, row, col, c0):
    nn = tx.shape[0]
    ne = te.shape[0]
    E = row.shape[0]
    C = 256
    txf = jnp.concatenate(
        [tx.reshape(nn * _TW), c0, jnp.zeros((4,), jnp.float32)])
    tef = te.reshape(ne * _TW)
    mesh = plsc.VectorSubcoreMesh(core_axis_name="c", subcore_axis_name="s")
    cp = pltpu.CompilerParams()
    if "needs_layout_passes" in pltpu.CompilerParams.__dataclass_fields__:
        cp = dataclasses.replace(cp, needs_layout_passes=False)

    @functools.partial(
        pl.kernel,
        compiler_params=cp,
        out_type=jax.ShapeDtypeStruct((_D, E), jnp.float32),
        mesh=mesh,
        scratch_types=[
            pltpu.VMEM((nn * _TW + 8,), jnp.float32),
            pltpu.VMEM((ne * _TW,), jnp.float32),
        ],
    )
    def sc_kernel(tx_hbm, te_hbm, row_hbm, col_hbm, out_hbm, txv, tev):
        pltpu.sync_copy(tx_hbm, txv)
        pltpu.sync_copy(te_hbm, tev)

        def body(rowv, colv, outv):
            @pl.loop(0, C, step=_L)
            def _vec(j):
                rb = rowv[pl.ds(j, _L)] * _TW
                cb = colv[pl.ds(j, _L)] * _TW

                def g(tab, ids, k):
                    return plsc.load_gather(tab, [ids + k])

                p = [g(txv, rb, k) + g(tev, cb, k) for k in range(_D)]
                ssum = g(txv, rb, 4) + g(tev, cb, 4)
                qsum = g(txv, rb, 5) + g(tev, cb, 5)
                mu = ssum * (1.0 / 256.0)
                v = qsum * (1.0 / 256.0) - mu * mu + _EPS
                iv = plsc.bitcast(v, jnp.int32)
                y = plsc.bitcast(jnp.int32(0x5F3759DF) - (iv >> 1), jnp.float32)
                y = y * (1.5 - 0.5 * v * y * y)
                y = y * (1.5 - 0.5 * v * y * y)
                y = y * (1.5 - 0.5 * v * y * y)
                for k in range(_D):
                    c0k = plsc.load_gather(
                        txv, [jnp.full((_L,), nn * _TW + k, jnp.int32)])
                    z = p[k] * y + c0k
                    o = 1.0 / (1.0 + jnp.exp(-z))
                    outv[k, pl.ds(j, _L)] = o

        pltpu.emit_pipeline(
            body,
            grid=(E // C,),
            in_specs=[pl.BlockSpec((C,), lambda i: (i,)),
                      pl.BlockSpec((C,), lambda i: (i,))],
            out_specs=[pl.BlockSpec((_D, C), lambda i: (0, i))],
            core_axis_name=("c", "s"),
            dimension_semantics=(pltpu.PARALLEL,),
        )(row_hbm, col_hbm, out_hbm)

    return sc_kernel(txf, tef, row, col).T


def kernel(x, e, hyperedge_index, node_types, hyperedge_types,
           ln_gamma, ln_beta, W, b):
    df = x.shape[-1]
    Wg = ln_gamma[:, None] * W
    gw = ln_gamma @ W
    c0 = ln_beta @ W + b
    ones = jnp.ones((df, 1), jnp.float32)
    zeros = jnp.zeros((df, 1), jnp.float32)
    wx = jnp.concatenate([Wg[:df] - gw[None, :] * (1.0 / 256.0), ones, zeros],
                         axis=1)
    we = jnp.concatenate([Wg[df:] - gw[None, :] * (1.0 / 256.0), ones, zeros],
                         axis=1)
    hi = hyperedge_index.astype(jnp.int32)
    tx, te, row, col = _build_tables(x, e, wx, we, hi)
    E = hi.shape[1]
    return _sc_combine(tx, te, row.reshape(E), col.reshape(E), c0)


# speedup vs baseline: 22.1046x; 1.5951x over previous
import dataclasses
import functools

import jax
import jax.numpy as jnp
from jax import lax
from jax.experimental import pallas as pl
from jax.experimental.pallas import tpu as pltpu
from jax.experimental.pallas import tpu_sc as plsc

_D = 4
_TW = 6
_EPS = 1e-5
_L = 16
_NC, _NS = 2, 16
_NW = _NC * _NS


def _tables_body(x_ref, e_ref, wx_ref, we_ref, hi_ref, tx_ref, te_ref,
                 row_ref, col_ref):
    df = wx_ref.shape[0]
    pid = pl.program_id(0)
    hi = hi_ref[...]
    row_ref[...] = hi[0, :][None, None, :]
    col_ref[...] = hi[1, :][None, None, :]

    def one(src, w, dst):
        m = (src[0::_D, :] + src[1::_D, :] + src[2::_D, :]
             + src[3::_D, :]) * 0.25
        a = jnp.dot(m, w[...], preferred_element_type=jnp.float32,
                    precision=lax.Precision.HIGHEST)
        q = jnp.sum(m * m, axis=1, keepdims=True)
        colid = lax.broadcasted_iota(jnp.int32, a.shape, 1)
        bn = a.shape[0]
        dst[pl.ds(pid * bn, bn), :] = a + jnp.where(colid == _TW - 1, q, 0.0)

    one(x_ref, wx_ref, tx_ref)
    one(e_ref, we_ref, te_ref)


def _build_tables(x, e, wx, we, hi):
    nn = x.shape[0] // _D
    ne = e.shape[0] // _D
    E = hi.shape[1]
    df = wx.shape[0]
    bn = 2000
    grid = (nn // bn,)
    be = E // grid[0]
    return pl.pallas_call(
        _tables_body,
        grid=grid,
        in_specs=[
            pl.BlockSpec((bn * _D, df), lambda i: (i, 0)),
            pl.BlockSpec((bn * _D, df), lambda i: (i, 0)),
            pl.BlockSpec((df, _TW), lambda i: (0, 0)),
            pl.BlockSpec((df, _TW), lambda i: (0, 0)),
            pl.BlockSpec((2, be), lambda i: (0, i)),
        ],
        out_specs=[
            pl.BlockSpec((nn, _TW), lambda i: (0, 0)),
            pl.BlockSpec((ne, _TW), lambda i: (0, 0)),
            pl.BlockSpec((1, 1, be), lambda i: (i, 0, 0)),
            pl.BlockSpec((1, 1, be), lambda i: (i, 0, 0)),
        ],
        out_shape=[
            jax.ShapeDtypeStruct((nn, _TW), jnp.float32),
            jax.ShapeDtypeStruct((ne, _TW), jnp.float32),
            jax.ShapeDtypeStruct((grid[0], 1, be), jnp.int32),
            jax.ShapeDtypeStruct((grid[0], 1, be), jnp.int32),
        ],
    )(x, e, wx, we, hi)


def _sc_combine(tx, te, row, col, c0):
    nn = tx.shape[0]
    ne = te.shape[0]
    E = row.shape[0]
    C = 512
    txf = jnp.concatenate(
        [tx.reshape(nn * _TW), c0, jnp.zeros((4,), jnp.float32)])
    tef = te.reshape(ne * _TW)
    mesh = plsc.VectorSubcoreMesh(core_axis_name="c", subcore_axis_name="s")
    cp = pltpu.CompilerParams()
    if "needs_layout_passes" in pltpu.CompilerParams.__dataclass_fields__:
        cp = dataclasses.replace(cp, needs_layout_passes=False)

    @functools.partial(
        pl.kernel,
        compiler_params=cp,
        out_type=jax.ShapeDtypeStruct((_D, E), jnp.float32),
        mesh=mesh,
        scratch_types=[
            pltpu.VMEM((nn * _TW + 8,), jnp.float32),
            pltpu.VMEM((ne * _TW,), jnp.float32),
        ],
    )
    def sc_kernel(tx_hbm, te_hbm, row_hbm, col_hbm, out_hbm, txv, tev):
        pltpu.sync_copy(tx_hbm, txv)
        pltpu.sync_copy(te_hbm, tev)

        def body(rowv, colv, outv):
            @plsc.parallel_loop(0, C, step=_L, unroll=4)
            def _vec(j):
                rb = rowv[pl.ds(j, _L)] * _TW
                cb = colv[pl.ds(j, _L)] * _TW

                def g(tab, ids, k):
                    return plsc.load_gather(tab, [ids + k])

                p = [g(txv, rb, k) + g(tev, cb, k) for k in range(_D)]
                ssum = g(txv, rb, 4) + g(tev, cb, 4)
                qsum = g(txv, rb, 5) + g(tev, cb, 5)
                mu = ssum * (1.0 / 256.0)
                v = qsum * (1.0 / 256.0) - mu * mu + _EPS
                iv = plsc.bitcast(v, jnp.int32)
                y = plsc.bitcast(jnp.int32(0x5F3759DF) - (iv >> 1), jnp.float32)
                y = y * (1.5 - 0.5 * v * y * y)
                y = y * (1.5 - 0.5 * v * y * y)
                for k in range(_D):
                    c0k = plsc.load_gather(
                        txv, [jnp.full((_L,), nn * _TW + k, jnp.int32)])
                    z = p[k] * y + c0k
                    o = 1.0 / (1.0 + jnp.exp(-z))
                    outv[k, pl.ds(j, _L)] = o

        pltpu.emit_pipeline(
            body,
            grid=(E // C,),
            in_specs=[pl.BlockSpec((C,), lambda i: (i,)),
                      pl.BlockSpec((C,), lambda i: (i,))],
            out_specs=[pl.BlockSpec((_D, C), lambda i: (0, i))],
            core_axis_name=("c", "s"),
            dimension_semantics=(pltpu.PARALLEL,),
        )(row_hbm, col_hbm, out_hbm)

    return sc_kernel(txf, tef, row, col).T


def kernel(x, e, hyperedge_index, node_types, hyperedge_types,
           ln_gamma, ln_beta, W, b):
    df = x.shape[-1]
    Wg = ln_gamma[:, None] * W
    gw = ln_gamma @ W
    c0 = ln_beta @ W + b
    ones = jnp.ones((df, 1), jnp.float32)
    zeros = jnp.zeros((df, 1), jnp.float32)
    wx = jnp.concatenate([Wg[:df] - gw[None, :] * (1.0 / 256.0), ones, zeros],
                         axis=1)
    we = jnp.concatenate([Wg[df:] - gw[None, :] * (1.0 / 256.0), ones, zeros],
                         axis=1)
    hi = hyperedge_index.astype(jnp.int32)
    tx, te, row, col = _build_tables(x, e, wx, we, hi)
    E = hi.shape[1]
    return _sc_combine(tx, te, row.reshape(E), col.reshape(E), c0)


# speedup vs baseline: 24.6208x; 1.1138x over previous
import dataclasses
import functools

import jax
import jax.numpy as jnp
from jax import lax
from jax.experimental import pallas as pl
from jax.experimental.pallas import tpu as pltpu
from jax.experimental.pallas import tpu_sc as plsc

_D = 4
_TW = 6
_EPS = 1e-5
_L = 16
_NC, _NS = 2, 16
_NW = _NC * _NS


def _tables_body(x_ref, e_ref, wx_ref, we_ref, tx_ref, te_ref):
    df = wx_ref.shape[0]
    pid = pl.program_id(0)

    def one(src, w, dst):
        m = (src[0::_D, :] + src[1::_D, :] + src[2::_D, :]
             + src[3::_D, :]) * 0.25
        a = jnp.dot(m, w[...], preferred_element_type=jnp.float32,
                    precision=lax.Precision.HIGHEST)
        q = jnp.sum(m * m, axis=1, keepdims=True)
        colid = lax.broadcasted_iota(jnp.int32, a.shape, 1)
        bn = a.shape[0]
        dst[pl.ds(pid * bn, bn), :] = a + jnp.where(colid == _TW - 1, q, 0.0)

    one(x_ref, wx_ref, tx_ref)
    one(e_ref, we_ref, te_ref)


def _build_tables(x, e, wx, we):
    nn = x.shape[0] // _D
    ne = e.shape[0] // _D
    df = wx.shape[0]
    bn = 1000
    grid = (nn // bn,)
    return pl.pallas_call(
        _tables_body,
        grid=grid,
        in_specs=[
            pl.BlockSpec((bn * _D, df), lambda i: (i, 0)),
            pl.BlockSpec((bn * _D, df), lambda i: (i, 0)),
            pl.BlockSpec((df, _TW), lambda i: (0, 0)),
            pl.BlockSpec((df, _TW), lambda i: (0, 0)),
        ],
        out_specs=[
            pl.BlockSpec((nn, _TW), lambda i: (0, 0)),
            pl.BlockSpec((ne, _TW), lambda i: (0, 0)),
        ],
        out_shape=[
            jax.ShapeDtypeStruct((nn, _TW), jnp.float32),
            jax.ShapeDtypeStruct((ne, _TW), jnp.float32),
        ],
    )(x, e, wx, we)


def _sc_combine(tx, te, hi, c0):
    nn = tx.shape[0]
    ne = te.shape[0]
    E = hi.shape[1]
    C = 512
    txf = tx.reshape(nn * _TW)
    tef = te.reshape(ne * _TW)
    c0p = jnp.concatenate([c0, jnp.zeros((4,), jnp.float32)])
    mesh = plsc.VectorSubcoreMesh(core_axis_name="c", subcore_axis_name="s")
    cp = pltpu.CompilerParams()
    if "needs_layout_passes" in pltpu.CompilerParams.__dataclass_fields__:
        cp = dataclasses.replace(cp, needs_layout_passes=False)

    @functools.partial(
        pl.kernel,
        compiler_params=cp,
        out_type=jax.ShapeDtypeStruct((_D, E), jnp.float32),
        mesh=mesh,
        scratch_types=[
            pltpu.VMEM((nn * _TW,), jnp.float32),
            pltpu.VMEM((ne * _TW,), jnp.float32),
            pltpu.VMEM((8,), jnp.float32),
            pltpu.SemaphoreType.DMA,
            pltpu.SemaphoreType.DMA,
        ],
    )
    def sc_kernel(tx_hbm, te_hbm, hi1_hbm, hi2_hbm, c0_hbm, out_hbm,
                  txv, tev, c0v, sem1, sem2):
        cp1 = pltpu.async_copy(tx_hbm, txv, sem1)
        cp2 = pltpu.async_copy(te_hbm, tev, sem2)
        pltpu.sync_copy(c0_hbm, c0v)
        cp1.wait()
        cp2.wait()

        def body(rowv, colv, outv):
            @plsc.parallel_loop(0, C, step=_L, unroll=4)
            def _vec(j):
                rb = rowv[0, pl.ds(j, _L)] * _TW
                cb = colv[0, pl.ds(j, _L)] * _TW

                def g(tab, ids, k):
                    return plsc.load_gather(tab, [ids + k])

                p = [g(txv, rb, k) + g(tev, cb, k) for k in range(_D)]
                ssum = g(txv, rb, 4) + g(tev, cb, 4)
                qsum = g(txv, rb, 5) + g(tev, cb, 5)
                mu = ssum * (1.0 / 256.0)
                v = qsum * (1.0 / 256.0) - mu * mu + _EPS
                iv = plsc.bitcast(v, jnp.int32)
                y = plsc.bitcast(jnp.int32(0x5F3759DF) - (iv >> 1), jnp.float32)
                y = y * (1.5 - 0.5 * v * y * y)
                y = y * (1.5 - 0.5 * v * y * y)
                for k in range(_D):
                    c0k = plsc.load_gather(
                        c0v, [jnp.full((_L,), k, jnp.int32)])
                    z = p[k] * y + c0k
                    o = 1.0 / (1.0 + jnp.exp(-z))
                    outv[k, pl.ds(j, _L)] = o

        pltpu.emit_pipeline(
            body,
            grid=(E // C,),
            in_specs=[pl.BlockSpec((1, C), lambda i: (0, i)),
                      pl.BlockSpec((1, C), lambda i: (1, i))],
            out_specs=[pl.BlockSpec((_D, C), lambda i: (0, i))],
            core_axis_name=("c", "s"),
            dimension_semantics=(pltpu.PARALLEL,),
        )(hi1_hbm, hi2_hbm, out_hbm)

    return sc_kernel(txf, tef, hi, hi, c0p).T


def kernel(x, e, hyperedge_index, node_types, hyperedge_types,
           ln_gamma, ln_beta, W, b):
    df = x.shape[-1]
    Wg = ln_gamma[:, None] * W
    gw = ln_gamma @ W
    c0 = ln_beta @ W + b
    ones = jnp.ones((df, 1), jnp.float32)
    zeros = jnp.zeros((df, 1), jnp.float32)
    wx = jnp.concatenate([Wg[:df] - gw[None, :] * (1.0 / 256.0), ones, zeros],
                         axis=1)
    we = jnp.concatenate([Wg[df:] - gw[None, :] * (1.0 / 256.0), ones, zeros],
                         axis=1)
    hi = hyperedge_index.astype(jnp.int32)
    tx, te = _build_tables(x, e, wx, we)
    return _sc_combine(tx, te, hi, c0)


# speedup vs baseline: 26.1942x; 1.0639x over previous
import dataclasses
import functools

import jax
import jax.numpy as jnp
from jax import lax
from jax.experimental import pallas as pl
from jax.experimental.pallas import tpu as pltpu
from jax.experimental.pallas import tpu_sc as plsc

_D = 4
_TW = 6
_FW = 750
_EPS = 1e-5
_L = 16
_NC, _NS = 2, 16
_NW = _NC * _NS


def _tables_body(x_ref, e_ref, wx_ref, we_ref, tx_ref, te_ref):
    df = wx_ref.shape[0]
    pid = pl.program_id(0)

    def one(src, w, dst):
        m = (src[0::_D, :] + src[1::_D, :] + src[2::_D, :]
             + src[3::_D, :]) * 0.25
        a = jnp.dot(m, w[...], preferred_element_type=jnp.float32,
                    precision=lax.Precision.HIGHEST)
        q = jnp.sum(m * m, axis=1, keepdims=True)
        colid = lax.broadcasted_iota(jnp.int32, a.shape, 1)
        bn = a.shape[0]
        dst[pl.ds(pid * bn, bn), :] = a + jnp.where(colid == _TW - 1, q, 0.0)

    one(x_ref, wx_ref, tx_ref)
    one(e_ref, we_ref, te_ref)


def _build_tables(x, e, wx, we):
    nn = x.shape[0] // _D
    ne = e.shape[0] // _D
    df = wx.shape[0]
    bn = 2000
    grid = (nn // bn,)
    return pl.pallas_call(
        _tables_body,
        grid=grid,
        in_specs=[
            pl.BlockSpec((bn * _D, df), lambda i: (i, 0)),
            pl.BlockSpec((bn * _D, df), lambda i: (i, 0)),
            pl.BlockSpec((df, _TW), lambda i: (0, 0)),
            pl.BlockSpec((df, _TW), lambda i: (0, 0)),
        ],
        out_specs=[
            pl.BlockSpec((nn, _TW), lambda i: (0, 0)),
            pl.BlockSpec((ne, _TW), lambda i: (0, 0)),
        ],
        out_shape=[
            jax.ShapeDtypeStruct((nn, _TW), jnp.float32),
            jax.ShapeDtypeStruct((ne, _TW), jnp.float32),
        ],
    )(x, e, wx, we)


def _sc_combine(tx, te, hi, c0):
    nn = tx.size // _TW
    ne = te.size // _TW
    E = hi.shape[1]
    C = 512
    txf = tx.reshape(nn * _TW)
    tef = te.reshape(ne * _TW)
    c0p = jnp.concatenate([c0, jnp.zeros((4,), jnp.float32)])
    mesh = plsc.VectorSubcoreMesh(core_axis_name="c", subcore_axis_name="s")
    cp = pltpu.CompilerParams()
    if "needs_layout_passes" in pltpu.CompilerParams.__dataclass_fields__:
        cp = dataclasses.replace(cp, needs_layout_passes=False)

    @functools.partial(
        pl.kernel,
        compiler_params=cp,
        out_type=jax.ShapeDtypeStruct((_D, E), jnp.float32),
        mesh=mesh,
        scratch_types=[
            pltpu.VMEM((nn * _TW,), jnp.float32),
            pltpu.VMEM((ne * _TW,), jnp.float32),
            pltpu.VMEM((8,), jnp.float32),
            pltpu.SemaphoreType.DMA,
            pltpu.SemaphoreType.DMA,
        ],
    )
    def sc_kernel(tx_hbm, te_hbm, hi1_hbm, hi2_hbm, c0_hbm, out_hbm,
                  txv, tev, c0v, sem1, sem2):
        cp1 = pltpu.async_copy(tx_hbm, txv, sem1)
        cp2 = pltpu.async_copy(te_hbm, tev, sem2)
        pltpu.sync_copy(c0_hbm, c0v)
        cp1.wait()
        cp2.wait()

        def body(rowv, colv, outv):
            @plsc.parallel_loop(0, C, step=_L, unroll=8)
            def _vec(j):
                rb = rowv[0, pl.ds(j, _L)] * _TW
                cb = colv[0, pl.ds(j, _L)] * _TW

                def g(tab, ids, k):
                    return plsc.load_gather(tab, [ids + k])

                p = [g(txv, rb, k) + g(tev, cb, k) for k in range(_D)]
                ssum = g(txv, rb, 4) + g(tev, cb, 4)
                qsum = g(txv, rb, 5) + g(tev, cb, 5)
                mu = ssum * (1.0 / 256.0)
                v = qsum * (1.0 / 256.0) - mu * mu + _EPS
                iv = plsc.bitcast(v, jnp.int32)
                y = plsc.bitcast(jnp.int32(0x5F3759DF) - (iv >> 1), jnp.float32)
                y = y * (1.5 - 0.5 * v * y * y)
                y = y * (1.5 - 0.5 * v * y * y)
                for k in range(_D):
                    c0k = plsc.load_gather(
                        c0v, [jnp.full((_L,), k, jnp.int32)])
                    z = p[k] * y + c0k
                    o = 1.0 / (1.0 + jnp.exp(-z))
                    outv[k, pl.ds(j, _L)] = o

        pltpu.emit_pipeline(
            body,
            grid=(E // C,),
            in_specs=[pl.BlockSpec((1, C), lambda i: (0, i)),
                      pl.BlockSpec((1, C), lambda i: (1, i))],
            out_specs=[pl.BlockSpec((_D, C), lambda i: (0, i))],
            core_axis_name=("c", "s"),
            dimension_semantics=(pltpu.PARALLEL,),
        )(hi1_hbm, hi2_hbm, out_hbm)

    return sc_kernel(txf, tef, hi, hi, c0p).T


def kernel(x, e, hyperedge_index, node_types, hyperedge_types,
           ln_gamma, ln_beta, W, b):
    df = x.shape[-1]
    Wg = ln_gamma[:, None] * W
    gw = ln_gamma @ W
    c0 = ln_beta @ W + b
    ones = jnp.ones((df, 1), jnp.float32)
    zeros = jnp.zeros((df, 1), jnp.float32)
    wx = jnp.concatenate([Wg[:df] - gw[None, :] * (1.0 / 256.0), ones, zeros],
                         axis=1)
    we = jnp.concatenate([Wg[df:] - gw[None, :] * (1.0 / 256.0), ones, zeros],
                         axis=1)
    hi = hyperedge_index.astype(jnp.int32)
    tx, te = _build_tables(x, e, wx, we)
    return _sc_combine(tx, te, hi, c0)
